# baseline (speedup 1.0000x reference)
import jax
import jax.numpy as jnp
from jax import lax
from jax.experimental import pallas as pl
from jax.experimental.pallas import tpu as pltpu
from jax.experimental.pallas import tpu_sc as plsc

VOCAB = 1000000
EMBED_DIM = 64
BATCH = 16384

NUM_CORES = 2
NUM_SUBCORES = 16
NUM_WORKERS = NUM_CORES * NUM_SUBCORES
B_PER_W = BATCH // NUM_WORKERS
LANES = 16
ROWCHUNK = 4
NCHUNK = B_PER_W // ROWCHUNK
IDX_PAD = B_PER_W + LANES


def _body(tgt_idx_hbm, ctx_idx_hbm, tgt_tab_hbm, ctx_tab_hbm, out_hbm,
          tidx_v, cidx_v, tbuf0, tbuf1, cbuf0, cbuf1, out_v,
          sem_t0, sem_t1, sem_c0, sem_c1):
    wid = lax.axis_index("s") * NUM_CORES + lax.axis_index("c")
    base = wid * B_PER_W

    pltpu.sync_copy(tgt_idx_hbm.at[pl.ds(base, B_PER_W)],
                    tidx_v.at[pl.ds(0, B_PER_W)])
    pltpu.sync_copy(ctx_idx_hbm.at[pl.ds(base, B_PER_W)],
                    cidx_v.at[pl.ds(0, B_PER_W)])
    zero16 = jnp.zeros((LANES,), jnp.int32)
    tidx_v[pl.ds(B_PER_W, LANES)] = zero16
    cidx_v[pl.ds(B_PER_W, LANES)] = zero16

    tbufs = (tbuf0, tbuf1)
    cbufs = (cbuf0, cbuf1)
    tsems = (sem_t0, sem_t1)
    csems = (sem_c0, sem_c1)

    def fire(ch, slot):
        sl = pl.ds(ch * ROWCHUNK, LANES)
        tvec = tidx_v[sl]
        cvec = cidx_v[sl]
        ttile = (tvec >> 7) << 7
        ctile = (cvec >> 7) << 7
        tsub = tvec & 127 & ~31
        csub = cvec & 127 & ~31
        for k in range(ROWCHUNK):
            tt = pl.multiple_of(ttile[k], 128)
            ct = pl.multiple_of(ctile[k], 128)
            ts = tsub[k]
            cs = csub[k]
            w = pl.ds(k * 32, 32)
            for sval in range(0, 128, 32):
                @pl.when(ts == sval)
                def _(sval=sval, tt=tt, w=w, slot=slot):
                    pltpu.make_async_copy(
                        tgt_tab_hbm.at[:, :, pl.ds(tt, 128)]
                        .at[:, :, pl.ds(sval, 32)],
                        tbufs[slot].at[:, :, w], tsems[slot]).start()

                @pl.when(cs == sval)
                def _(sval=sval, ct=ct, w=w, slot=slot):
                    pltpu.make_async_copy(
                        ctx_tab_hbm.at[:, :, pl.ds(ct, 128)]
                        .at[:, :, pl.ds(sval, 32)],
                        cbufs[slot].at[:, :, w], csems[slot]).start()

    def drain(slot):
        pltpu.make_async_copy(
            tgt_tab_hbm.at[:, :, pl.ds(0, 128)],
            tbufs[slot], tsems[slot]).wait()
        pltpu.make_async_copy(
            ctx_tab_hbm.at[:, :, pl.ds(0, 128)],
            cbufs[slot], csems[slot]).wait()

    def compute(ch, slot):
        sl = pl.ds(ch * ROWCHUNK, LANES)
        tvec = tidx_v[sl]
        cvec = cidx_v[sl]
        ii = lax.iota(jnp.int32, LANES)
        row = ii & 3
        feat = ii >> 2
        twin = (tvec & 31).at[row].get(mode="promise_in_bounds")
        cwin = (cvec & 31).at[row].get(mode="promise_in_bounds")
        tslot = (row << 5) + twin
        cslot = (row << 5) + cwin
        tb = tbufs[slot]
        cb = cbufs[slot]
        acc = jnp.zeros((LANES,), jnp.float32)
        for c0 in range(0, EMBED_DIM, 4):
            a0 = jnp.full((LANES,), c0 // 8, jnp.int32)
            b0 = (c0 % 8) + feat
            tv = plsc.load_gather(tb, [a0, b0, tslot])
            cv = plsc.load_gather(cb, [a0, b0, cslot])
            acc = acc + tv * cv
        acc = acc + acc.at[ii ^ 8].get(mode="promise_in_bounds")
        acc = acc + acc.at[ii ^ 4].get(mode="promise_in_bounds")
        plsc.store_scatter(out_v, [ch * ROWCHUNK + row], acc,
                           mask=ii < ROWCHUNK)

    fire(0, 0)

    def two_chunks(j, _):
        ch0 = j * 2
        fire(ch0 + 1, 1)
        drain(0)
        compute(ch0, 0)

        @pl.when(j < NCHUNK // 2 - 1)
        def _():
            fire(ch0 + 2, 0)

        drain(1)
        compute(ch0 + 1, 1)
        return 0

    lax.fori_loop(0, NCHUNK // 2, two_chunks, 0)

    pltpu.sync_copy(out_v, out_hbm.at[pl.ds(base, B_PER_W)])


@jax.jit
def _run(target, context, target_table, context_table):
    mesh = plsc.VectorSubcoreMesh(core_axis_name="c", subcore_axis_name="s")
    kfn = pl.kernel(
        _body,
        mesh=mesh,
        compiler_params=pltpu.CompilerParams(
            needs_layout_passes=False, use_tc_tiling_on_sc=True),
        out_type=jax.ShapeDtypeStruct((BATCH,), jnp.float32),
        scratch_types=[
            pltpu.VMEM((IDX_PAD,), jnp.int32),
            pltpu.VMEM((IDX_PAD,), jnp.int32),
            pltpu.VMEM((8, 8, 128), jnp.float32),
            pltpu.VMEM((8, 8, 128), jnp.float32),
            pltpu.VMEM((8, 8, 128), jnp.float32),
            pltpu.VMEM((8, 8, 128), jnp.float32),
            pltpu.VMEM((B_PER_W,), jnp.float32),
            pltpu.SemaphoreType.DMA,
            pltpu.SemaphoreType.DMA,
            pltpu.SemaphoreType.DMA,
            pltpu.SemaphoreType.DMA,
        ],
    )
    tt = target_table.T.reshape(8, 8, VOCAB)
    ct = context_table.T.reshape(8, 8, VOCAB)
    return kfn(target, context, tt, ct)


def kernel(target, context, target_table, context_table):
    return _run(target.astype(jnp.int32), context.astype(jnp.int32),
                target_table, context_table)


# speedup vs baseline: 1.3919x; 1.3919x over previous
import jax
import jax.numpy as jnp
from jax import lax
from jax.experimental import pallas as pl
from jax.experimental.pallas import tpu as pltpu
from jax.experimental.pallas import tpu_sc as plsc

VOCAB = 1000000
EMBED_DIM = 64
BATCH = 16384

NUM_CORES = 2
NUM_SUBCORES = 16
NUM_WORKERS = NUM_CORES * NUM_SUBCORES
B_PER_W = BATCH // NUM_WORKERS
LANES = 16
ROWCHUNK = 4
NCHUNK = B_PER_W // ROWCHUNK
IDX_PAD = B_PER_W + LANES


def _body(tgt_idx_hbm, ctx_idx_hbm, tgt_tab_hbm, ctx_tab_hbm, out_hbm,
          tidx_v, cidx_v, tbuf0, tbuf1, tbuf2, cbuf0, cbuf1, cbuf2, out_v,
          sem_t0, sem_t1, sem_t2, sem_c0, sem_c1, sem_c2):
    wid = lax.axis_index("s") * NUM_CORES + lax.axis_index("c")
    base = wid * B_PER_W

    pltpu.sync_copy(tgt_idx_hbm.at[pl.ds(base, B_PER_W)],
                    tidx_v.at[pl.ds(0, B_PER_W)])
    pltpu.sync_copy(ctx_idx_hbm.at[pl.ds(base, B_PER_W)],
                    cidx_v.at[pl.ds(0, B_PER_W)])
    zero16 = jnp.zeros((LANES,), jnp.int32)
    tidx_v[pl.ds(B_PER_W, LANES)] = zero16
    cidx_v[pl.ds(B_PER_W, LANES)] = zero16

    tbufs = (tbuf0, tbuf1, tbuf2)
    cbufs = (cbuf0, cbuf1, cbuf2)
    tsems = (sem_t0, sem_t1, sem_t2)
    csems = (sem_c0, sem_c1, sem_c2)

    def fire(ch, slot):
        sl = pl.ds(ch * ROWCHUNK, LANES)
        tvec = tidx_v[sl]
        cvec = cidx_v[sl]
        ttile = (tvec >> 7) << 7
        ctile = (cvec >> 7) << 7
        tsub = tvec & 127 & ~63
        csub = cvec & 127 & ~63
        for k in range(ROWCHUNK):
            tt = pl.multiple_of(ttile[k], 128)
            ct = pl.multiple_of(ctile[k], 128)
            ts = tsub[k]
            cs = csub[k]
            w = pl.ds(k * 64, 64)
            for sval in range(0, 128, 64):
                @pl.when(ts == sval)
                def _(sval=sval, tt=tt, w=w, slot=slot):
                    pltpu.make_async_copy(
                        tgt_tab_hbm.at[:, :, pl.ds(tt, 128)]
                        .at[:, :, pl.ds(sval, 64)],
                        tbufs[slot].at[:, :, w], tsems[slot]).start()

                @pl.when(cs == sval)
                def _(sval=sval, ct=ct, w=w, slot=slot):
                    pltpu.make_async_copy(
                        ctx_tab_hbm.at[:, :, pl.ds(ct, 128)]
                        .at[:, :, pl.ds(sval, 64)],
                        cbufs[slot].at[:, :, w], csems[slot]).start()

    def drain(slot):
        pltpu.make_async_copy(
            tgt_tab_hbm.at[:, :, pl.ds(0, 256)],
            tbufs[slot], tsems[slot]).wait()
        pltpu.make_async_copy(
            ctx_tab_hbm.at[:, :, pl.ds(0, 256)],
            cbufs[slot], csems[slot]).wait()

    def compute(ch, slot):
        sl = pl.ds(ch * ROWCHUNK, LANES)
        tvec = tidx_v[sl]
        cvec = cidx_v[sl]
        ii = lax.iota(jnp.int32, LANES)
        row = ii & 3
        feat = ii >> 2
        twin = (tvec & 63).at[row].get(mode="promise_in_bounds")
        cwin = (cvec & 63).at[row].get(mode="promise_in_bounds")
        tslot = (row << 6) + twin
        cslot = (row << 6) + cwin
        tb = tbufs[slot]
        cb = cbufs[slot]
        acc = jnp.zeros((LANES,), jnp.float32)
        for c0 in range(0, EMBED_DIM, 4):
            a0 = jnp.full((LANES,), c0 // 8, jnp.int32)
            b0 = (c0 % 8) + feat
            tv = plsc.load_gather(tb, [a0, b0, tslot])
            cv = plsc.load_gather(cb, [a0, b0, cslot])
            acc = acc + tv * cv
        acc = acc + acc.at[ii ^ 8].get(mode="promise_in_bounds")
        acc = acc + acc.at[ii ^ 4].get(mode="promise_in_bounds")
        plsc.store_scatter(out_v, [ch * ROWCHUNK + row], acc,
                           mask=ii < ROWCHUNK)

    fire(0, 0)
    fire(1, 1)

    def three_chunks(j, _):
        ch = j * 3
        fire(ch + 2, 2)
        drain(0)
        compute(ch, 0)

        @pl.when(ch + 3 < NCHUNK)
        def _():
            fire(ch + 3, 0)

        drain(1)
        compute(ch + 1, 1)

        @pl.when(ch + 4 < NCHUNK)
        def _():
            fire(ch + 4, 1)

        drain(2)
        compute(ch + 2, 2)
        return 0

    lax.fori_loop(0, (NCHUNK - 2) // 3, three_chunks, 0)
    drain(0)
    compute(NCHUNK - 2, 0)
    drain(1)
    compute(NCHUNK - 1, 1)

    pltpu.sync_copy(out_v, out_hbm.at[pl.ds(base, B_PER_W)])


@jax.jit
def _run(target, context, target_table, context_table):
    mesh = plsc.VectorSubcoreMesh(core_axis_name="c", subcore_axis_name="s")
    kfn = pl.kernel(
        _body,
        mesh=mesh,
        compiler_params=pltpu.CompilerParams(
            needs_layout_passes=False, use_tc_tiling_on_sc=True),
        out_type=jax.ShapeDtypeStruct((BATCH,), jnp.float32),
        scratch_types=[
            pltpu.VMEM((IDX_PAD,), jnp.int32),
            pltpu.VMEM((IDX_PAD,), jnp.int32),
            pltpu.VMEM((8, 8, 256), jnp.float32),
            pltpu.VMEM((8, 8, 256), jnp.float32),
            pltpu.VMEM((8, 8, 256), jnp.float32),
            pltpu.VMEM((8, 8, 256), jnp.float32),
            pltpu.VMEM((8, 8, 256), jnp.float32),
            pltpu.VMEM((8, 8, 256), jnp.float32),
            pltpu.VMEM((B_PER_W,), jnp.float32),
            pltpu.SemaphoreType.DMA,
            pltpu.SemaphoreType.DMA,
            pltpu.SemaphoreType.DMA,
            pltpu.SemaphoreType.DMA,
            pltpu.SemaphoreType.DMA,
            pltpu.SemaphoreType.DMA,
        ],
    )
    tt = target_table.T.reshape(8, 8, VOCAB)
    ct = context_table.T.reshape(8, 8, VOCAB)
    return kfn(target, context, tt, ct)


def kernel(target, context, target_table, context_table):
    return _run(target.astype(jnp.int32), context.astype(jnp.int32),
                target_table, context_table)


# speedup vs baseline: 1.4548x; 1.0452x over previous
import jax
import jax.numpy as jnp
from jax import lax
from jax.experimental import pallas as pl
from jax.experimental.pallas import tpu as pltpu
from jax.experimental.pallas import tpu_sc as plsc

VOCAB = 1000000
EMBED_DIM = 64
BATCH = 16384

NUM_CORES = 2
NUM_SUBCORES = 16
NUM_WORKERS = NUM_CORES * NUM_SUBCORES
B_PER_W = BATCH // NUM_WORKERS
LANES = 16
ROWCHUNK = 4
NCHUNK = B_PER_W // ROWCHUNK
IDX_PAD = B_PER_W + LANES


def _body(tgt_idx_hbm, ctx_idx_hbm, tgt_tab_hbm, ctx_tab_hbm, out_hbm,
          tidx_v, cidx_v, tbuf0, tbuf1, cbuf0, cbuf1, out_v,
          sem_t0, sem_t1, sem_c0, sem_c1):
    wid = lax.axis_index("s") * NUM_CORES + lax.axis_index("c")
    base = wid * B_PER_W

    pltpu.sync_copy(tgt_idx_hbm.at[pl.ds(base, B_PER_W)],
                    tidx_v.at[pl.ds(0, B_PER_W)])
    pltpu.sync_copy(ctx_idx_hbm.at[pl.ds(base, B_PER_W)],
                    cidx_v.at[pl.ds(0, B_PER_W)])
    zero16 = jnp.zeros((LANES,), jnp.int32)
    tidx_v[pl.ds(B_PER_W, LANES)] = zero16
    cidx_v[pl.ds(B_PER_W, LANES)] = zero16

    tbufs = (tbuf0, tbuf1)
    cbufs = (cbuf0, cbuf1)
    tsems = (sem_t0, sem_t1)
    csems = (sem_c0, sem_c1)

    def fire(ch, slot):
        sl = pl.ds(ch * ROWCHUNK, LANES)
        tvec = tidx_v[sl]
        cvec = cidx_v[sl]
        ttile = (tvec >> 7) << 7
        ctile = (cvec >> 7) << 7
        tsub = tvec & 127 & ~63
        csub = cvec & 127 & ~63
        for k in range(ROWCHUNK):
            tt = pl.multiple_of(ttile[k], 128)
            ct = pl.multiple_of(ctile[k], 128)
            ts = tsub[k]
            cs = csub[k]
            w = pl.ds(k * 64, 64)
            for sval in range(0, 128, 64):
                @pl.when(ts == sval)
                def _(sval=sval, tt=tt, w=w, slot=slot):
                    pltpu.make_async_copy(
                        tgt_tab_hbm.at[:, :, pl.ds(tt, 128)]
                        .at[:, :, pl.ds(sval, 64)],
                        tbufs[slot].at[:, :, w], tsems[slot]).start()

                @pl.when(cs == sval)
                def _(sval=sval, ct=ct, w=w, slot=slot):
                    pltpu.make_async_copy(
                        ctx_tab_hbm.at[:, :, pl.ds(ct, 128)]
                        .at[:, :, pl.ds(sval, 64)],
                        cbufs[slot].at[:, :, w], csems[slot]).start()

    def drain(slot):
        pltpu.make_async_copy(
            tgt_tab_hbm.at[:, :, pl.ds(0, 256)],
            tbufs[slot], tsems[slot]).wait()
        pltpu.make_async_copy(
            ctx_tab_hbm.at[:, :, pl.ds(0, 256)],
            cbufs[slot], csems[slot]).wait()

    def compute(ch, slot):
        sl = pl.ds(ch * ROWCHUNK, LANES)
        tvec = tidx_v[sl]
        cvec = cidx_v[sl]
        ii = lax.iota(jnp.int32, LANES)
        row = ii & 3
        feat = ii >> 2
        twin = (tvec & 63).at[row].get(mode="promise_in_bounds")
        cwin = (cvec & 63).at[row].get(mode="promise_in_bounds")
        tslot = (row << 6) + twin
        cslot = (row << 6) + cwin
        tb = tbufs[slot]
        cb = cbufs[slot]
        acc = jnp.zeros((LANES,), jnp.float32)
        for c0 in range(0, EMBED_DIM, 4):
            a0 = jnp.full((LANES,), c0 // 8, jnp.int32)
            b0 = (c0 % 8) + feat
            tv = plsc.load_gather(tb, [a0, b0, tslot])
            cv = plsc.load_gather(cb, [a0, b0, cslot])
            acc = acc + tv * cv
        acc = acc + acc.at[ii ^ 8].get(mode="promise_in_bounds")
        acc = acc + acc.at[ii ^ 4].get(mode="promise_in_bounds")
        plsc.store_scatter(out_v, [ch * ROWCHUNK + row], acc,
                           mask=ii < ROWCHUNK)

    fire(0, 0)

    def two_chunks(j, _):
        ch0 = j * 2
        fire(ch0 + 1, 1)
        drain(0)
        compute(ch0, 0)

        @pl.when(j < NCHUNK // 2 - 1)
        def _():
            fire(ch0 + 2, 0)

        drain(1)
        compute(ch0 + 1, 1)
        return 0

    lax.fori_loop(0, NCHUNK // 2, two_chunks, 0)

    pltpu.sync_copy(out_v, out_hbm.at[pl.ds(base, B_PER_W)])


@jax.jit
def _run(target, context, target_table, context_table):
    mesh = plsc.VectorSubcoreMesh(core_axis_name="c", subcore_axis_name="s")
    kfn = pl.kernel(
        _body,
        mesh=mesh,
        compiler_params=pltpu.CompilerParams(
            needs_layout_passes=False, use_tc_tiling_on_sc=True),
        out_type=jax.ShapeDtypeStruct((BATCH,), jnp.float32),
        scratch_types=[
            pltpu.VMEM((IDX_PAD,), jnp.int32),
            pltpu.VMEM((IDX_PAD,), jnp.int32),
            pltpu.VMEM((8, 8, 256), jnp.float32),
            pltpu.VMEM((8, 8, 256), jnp.float32),
            pltpu.VMEM((8, 8, 256), jnp.float32),
            pltpu.VMEM((8, 8, 256), jnp.float32),
            pltpu.VMEM((B_PER_W,), jnp.float32),
            pltpu.SemaphoreType.DMA,
            pltpu.SemaphoreType.DMA,
            pltpu.SemaphoreType.DMA,
            pltpu.SemaphoreType.DMA,
        ],
    )
    tt = target_table.T.reshape(8, 8, VOCAB)
    ct = context_table.T.reshape(8, 8, VOCAB)
    return kfn(target, context, tt, ct)


def kernel(target, context, target_table, context_table):
    return _run(target.astype(jnp.int32), context.astype(jnp.int32),
                target_table, context_table)
